# baseline (speedup 1.0000x reference)
import functools

import jax
import jax.numpy as jnp
from jax import lax
from jax.experimental import pallas as pl
from jax.experimental.pallas import tpu as pltpu
from jax.experimental.pallas import tpu_sc as plsc

GRID_W = 64
N_NEURONS = 4096
B = 1024
D = 32

_NC = 2
_NS = 16
_NW = _NC * _NS
_RPW = B // _NW
_PAD = 128


def _bf16_split_dot(a, b):
    ah = a.astype(jnp.bfloat16)
    al = (a - ah.astype(jnp.float32)).astype(jnp.bfloat16)
    bh = b.astype(jnp.bfloat16)
    bl = (b - bh.astype(jnp.float32)).astype(jnp.bfloat16)
    a3 = jnp.concatenate([ah, ah, al], axis=1)
    b3 = jnp.concatenate([bh, bl, bh], axis=0)
    return jax.lax.dot_general(
        a3, b3, (((1,), (0,)), ((), ())), preferred_element_type=jnp.float32)


def _top2_body(x_ref, wt_ref, i1_ref, i2_ref, i1c_ref, i2c_ref):
    x = x_ref[:, :]
    wt = wt_ref[:, :]

    xw = _bf16_split_dot(x, wt)
    wn = jnp.sum(wt * wt, axis=0)
    score = wn[None, :] - 2.0 * xw

    col = jax.lax.broadcasted_iota(jnp.int32, score.shape, 1)
    i1 = jnp.argmin(score, axis=1).astype(jnp.int32)
    masked = jnp.where(col == i1[:, None], jnp.inf, score)
    i2 = jnp.argmin(masked, axis=1).astype(jnp.int32)

    i1_ref[:] = i1
    i2_ref[:] = i2
    i1c_ref[:, 0] = i1
    i2c_ref[:, 0] = i2


@functools.lru_cache(maxsize=1)
def _make_sc_gather():
    mesh = plsc.VectorSubcoreMesh(
        core_axis_name="c", subcore_axis_name="s",
        num_cores=_NC, num_subcores=_NS,
    )

    @functools.partial(
        pl.kernel,
        out_type=(
            jax.ShapeDtypeStruct((B, _PAD), jnp.float32),
            jax.ShapeDtypeStruct((B, _PAD), jnp.float32),
        ),
        mesh=mesh,
        compiler_params=pltpu.CompilerParams(use_tc_tiling_on_sc=False),
        scratch_types=[
            pltpu.VMEM((_RPW,), jnp.int32),
            pltpu.VMEM((_RPW,), jnp.int32),
            pltpu.VMEM((_RPW, D), jnp.float32),
            pltpu.VMEM((_RPW, D), jnp.float32),
            pltpu.VMEM((_RPW, _PAD), jnp.float32),
            pltpu.VMEM((_RPW, _PAD), jnp.float32),
            pltpu.SemaphoreType.DMA,
        ],
    )
    def _sc_gather(w_hbm, i1_hbm, i2_hbm, w1_hbm, w2_hbm,
                   i1_v, i2_v, w1_v, w2_v, w1p_v, w2p_v, sem):
        wid = lax.axis_index("s") * _NC + lax.axis_index("c")
        base = wid * _RPW
        pltpu.sync_copy(i1_hbm.at[pl.ds(base, _RPW)], i1_v)
        pltpu.sync_copy(i2_hbm.at[pl.ds(base, _RPW)], i2_v)
        cp1 = pltpu.async_copy(w_hbm.at[i1_v], w1_v, sem)
        cp2 = pltpu.async_copy(w_hbm.at[i2_v], w2_v, sem)
        cp1.wait()
        cp2.wait()
        for r in range(_RPW):
            for h in range(D // 16):
                chunk = pl.ds(h * 16, 16)
                w1p_v[r, chunk] = w1_v[r, chunk]
                w2p_v[r, chunk] = w2_v[r, chunk]
        pltpu.sync_copy(w1p_v, w1_hbm.at[pl.ds(base, _RPW)])
        pltpu.sync_copy(w2p_v, w2_hbm.at[pl.ds(base, _RPW)])

    return _sc_gather


def _finish_body(x_ref, w1_ref, w2_ref, i1_ref, i2_ref, rc_ref, qe_ref):
    x = x_ref[:, :]
    dd1 = x - w1_ref[:, pl.ds(0, D)]
    dd2 = x - w2_ref[:, pl.ds(0, D)]
    e1 = jnp.sum(dd1 * dd1, axis=1)
    e2 = jnp.sum(dd2 * dd2, axis=1)
    i1 = i1_ref[:, 0]
    i2 = i2_ref[:, 0]
    use2 = (e2 < e1) | ((e2 == e1) & (i2 < i1))
    bmu = jnp.where(use2, i2, i1)
    rc_ref[:, 0] = bmu // GRID_W
    rc_ref[:, 1] = bmu % GRID_W
    qe_ref[:, 0] = jnp.sqrt(jnp.where(use2, e2, e1))


def kernel(x, weights_map):
    w_flat = jnp.reshape(weights_map, (N_NEURONS, D))
    wt = jnp.transpose(w_flat)
    i1, i2, i1c, i2c = pl.pallas_call(
        _top2_body,
        out_shape=(
            jax.ShapeDtypeStruct((B,), jnp.int32),
            jax.ShapeDtypeStruct((B,), jnp.int32),
            jax.ShapeDtypeStruct((B, 1), jnp.int32),
            jax.ShapeDtypeStruct((B, 1), jnp.int32),
        ),
    )(x, wt)
    w1p, w2p = _make_sc_gather()(w_flat, i1, i2)
    rc, qe = pl.pallas_call(
        _finish_body,
        out_shape=(
            jax.ShapeDtypeStruct((B, 2), jnp.int32),
            jax.ShapeDtypeStruct((B, 1), jnp.float32),
        ),
    )(x, w1p, w2p, i1c, i2c)
    return rc, qe[:, 0]


# speedup vs baseline: 1.0213x; 1.0213x over previous
import functools

import jax
import jax.numpy as jnp
from jax import lax
from jax.experimental import pallas as pl
from jax.experimental.pallas import tpu as pltpu
from jax.experimental.pallas import tpu_sc as plsc

GRID_W = 64
N_NEURONS = 4096
B = 1024
D = 32

_NC = 2
_NS = 16
_NW = _NC * _NS
_RPW = B // _NW
_PAD = 128


def _bf16_split_dot(a, b):
    ah = a.astype(jnp.bfloat16)
    al = (a - ah.astype(jnp.float32)).astype(jnp.bfloat16)
    bh = b.astype(jnp.bfloat16)
    bl = (b - bh.astype(jnp.float32)).astype(jnp.bfloat16)
    a3 = jnp.concatenate([ah, ah, al], axis=1)
    b3 = jnp.concatenate([bh, bl, bh], axis=0)
    return jax.lax.dot_general(
        a3, b3, (((1,), (0,)), ((), ())), preferred_element_type=jnp.float32)


def _top2_body(x_ref, wt_ref, i1_ref, i2_ref):
    x = x_ref[:, :]
    wt = wt_ref[:, :]

    xw = _bf16_split_dot(x, wt)
    wn = jnp.sum(wt * wt, axis=0)
    score = wn[None, :] - 2.0 * xw

    col = jax.lax.broadcasted_iota(jnp.int32, score.shape, 1)
    i1 = jnp.argmin(score, axis=1).astype(jnp.int32)
    masked = jnp.where(col == i1[:, None], jnp.inf, score)
    i2 = jnp.argmin(masked, axis=1).astype(jnp.int32)

    i1_ref[:] = i1
    i2_ref[:] = i2


@functools.lru_cache(maxsize=1)
def _make_sc_gather():
    mesh = plsc.VectorSubcoreMesh(
        core_axis_name="c", subcore_axis_name="s",
        num_cores=_NC, num_subcores=_NS,
    )

    @functools.partial(
        pl.kernel,
        out_type=(
            jax.ShapeDtypeStruct((B, _PAD), jnp.float32),
            jax.ShapeDtypeStruct((B, _PAD), jnp.float32),
        ),
        mesh=mesh,
        compiler_params=pltpu.CompilerParams(use_tc_tiling_on_sc=False),
        scratch_types=[
            pltpu.VMEM((_RPW,), jnp.int32),
            pltpu.VMEM((_RPW,), jnp.int32),
            pltpu.VMEM((_RPW, D), jnp.float32),
            pltpu.VMEM((_RPW, D), jnp.float32),
            pltpu.VMEM((_RPW, _PAD), jnp.float32),
            pltpu.VMEM((_RPW, _PAD), jnp.float32),
            pltpu.SemaphoreType.DMA,
        ],
    )
    def _sc_gather(w_hbm, i1_hbm, i2_hbm, w1_hbm, w2_hbm,
                   i1_v, i2_v, w1_v, w2_v, w1p_v, w2p_v, sem):
        wid = lax.axis_index("s") * _NC + lax.axis_index("c")
        base = wid * _RPW
        pltpu.sync_copy(i1_hbm.at[pl.ds(base, _RPW)], i1_v)
        pltpu.sync_copy(i2_hbm.at[pl.ds(base, _RPW)], i2_v)
        cp1 = pltpu.async_copy(w_hbm.at[i1_v], w1_v, sem)
        cp2 = pltpu.async_copy(w_hbm.at[i2_v], w2_v, sem)
        cp1.wait()
        cp2.wait()
        for r in range(_RPW):
            for h in range(D // 16):
                chunk = pl.ds(h * 16, 16)
                w1p_v[r, chunk] = w1_v[r, chunk]
                w2p_v[r, chunk] = w2_v[r, chunk]
        pltpu.sync_copy(w1p_v, w1_hbm.at[pl.ds(base, _RPW)])
        pltpu.sync_copy(w2p_v, w2_hbm.at[pl.ds(base, _RPW)])

    return _sc_gather


def _finish_body(x_ref, w1_ref, w2_ref, i1_ref, i2_ref, rc_ref, qe_ref):
    x = x_ref[:, :]
    dd1 = x - w1_ref[:, pl.ds(0, D)]
    dd2 = x - w2_ref[:, pl.ds(0, D)]
    e1 = jnp.sum(dd1 * dd1, axis=1)
    e2 = jnp.sum(dd2 * dd2, axis=1)
    i1 = i1_ref[:]
    i2 = i2_ref[:]
    use2 = (e2 < e1) | ((e2 == e1) & (i2 < i1))
    bmu = jnp.where(use2, i2, i1)
    rc_ref[:, 0] = bmu // GRID_W
    rc_ref[:, 1] = bmu % GRID_W
    qe_ref[:, 0] = jnp.sqrt(jnp.where(use2, e2, e1))


def kernel(x, weights_map):
    w_flat = jnp.reshape(weights_map, (N_NEURONS, D))
    wt = jnp.transpose(w_flat)
    i1, i2 = pl.pallas_call(
        _top2_body,
        out_shape=(
            jax.ShapeDtypeStruct((B,), jnp.int32),
            jax.ShapeDtypeStruct((B,), jnp.int32),
        ),
    )(x, wt)
    w1p, w2p = _make_sc_gather()(w_flat, i1, i2)
    rc, qe = pl.pallas_call(
        _finish_body,
        out_shape=(
            jax.ShapeDtypeStruct((B, 2), jnp.int32),
            jax.ShapeDtypeStruct((B, 1), jnp.float32),
        ),
    )(x, w1p, w2p, i1, i2)
    return rc, qe[:, 0]


# speedup vs baseline: 1.1065x; 1.0834x over previous
import functools

import jax
import jax.numpy as jnp
from jax import lax
from jax.experimental import pallas as pl
from jax.experimental.pallas import tpu as pltpu
from jax.experimental.pallas import tpu_sc as plsc

GRID_W = 64
N_NEURONS = 4096
B = 1024
D = 32

_NC = 2
_NS = 16
_NW = _NC * _NS
_RPW = B // _NW
_PAD = 128


def _bf16_split_dot(a, b):
    ah = a.astype(jnp.bfloat16)
    al = (a - ah.astype(jnp.float32)).astype(jnp.bfloat16)
    bh = b.astype(jnp.bfloat16)
    bl = (b - bh.astype(jnp.float32)).astype(jnp.bfloat16)
    a3 = jnp.concatenate([ah, ah, al], axis=1)
    b3 = jnp.concatenate([bh, bl, bh], axis=0)
    return jax.lax.dot_general(
        a3, b3, (((1,), (0,)), ((), ())), preferred_element_type=jnp.float32)


def _top2_body(x_ref, w_ref, i1_ref, i2_ref):
    x = x_ref[:, :]
    wt = jnp.transpose(w_ref[:, :])

    xw = _bf16_split_dot(x, wt)
    wn = jnp.sum(wt * wt, axis=0)
    score = wn[None, :] - 2.0 * xw

    col = jax.lax.broadcasted_iota(jnp.int32, score.shape, 1)
    i1 = jnp.argmin(score, axis=1).astype(jnp.int32)
    masked = jnp.where(col == i1[:, None], jnp.inf, score)
    i2 = jnp.argmin(masked, axis=1).astype(jnp.int32)

    i1_ref[:] = i1
    i2_ref[:] = i2


@functools.lru_cache(maxsize=1)
def _make_sc_gather():
    mesh = plsc.VectorSubcoreMesh(
        core_axis_name="c", subcore_axis_name="s",
        num_cores=_NC, num_subcores=_NS,
    )

    @functools.partial(
        pl.kernel,
        out_type=(
            jax.ShapeDtypeStruct((B, _PAD), jnp.float32),
            jax.ShapeDtypeStruct((B, _PAD), jnp.float32),
        ),
        mesh=mesh,
        compiler_params=pltpu.CompilerParams(use_tc_tiling_on_sc=False),
        scratch_types=[
            pltpu.VMEM((_RPW,), jnp.int32),
            pltpu.VMEM((_RPW,), jnp.int32),
            pltpu.VMEM((_RPW, D), jnp.float32),
            pltpu.VMEM((_RPW, D), jnp.float32),
            pltpu.VMEM((_RPW, _PAD), jnp.float32),
            pltpu.VMEM((_RPW, _PAD), jnp.float32),
            pltpu.SemaphoreType.DMA,
        ],
    )
    def _sc_gather(w_hbm, i1_hbm, i2_hbm, w1_hbm, w2_hbm,
                   i1_v, i2_v, w1_v, w2_v, w1p_v, w2p_v, sem):
        wid = lax.axis_index("s") * _NC + lax.axis_index("c")
        base = wid * _RPW
        pltpu.sync_copy(i1_hbm.at[pl.ds(base, _RPW)], i1_v)
        pltpu.sync_copy(i2_hbm.at[pl.ds(base, _RPW)], i2_v)
        cp1 = pltpu.async_copy(w_hbm.at[i1_v], w1_v, sem)
        cp2 = pltpu.async_copy(w_hbm.at[i2_v], w2_v, sem)
        cp1.wait()
        cp2.wait()
        for r in range(_RPW):
            for h in range(D // 16):
                chunk = pl.ds(h * 16, 16)
                w1p_v[r, chunk] = w1_v[r, chunk]
                w2p_v[r, chunk] = w2_v[r, chunk]
        pltpu.sync_copy(w1p_v, w1_hbm.at[pl.ds(base, _RPW)])
        pltpu.sync_copy(w2p_v, w2_hbm.at[pl.ds(base, _RPW)])

    return _sc_gather


def _finish_body(x_ref, w1_ref, w2_ref, i1_ref, i2_ref, rc_ref, qe_ref):
    x = x_ref[:, :]
    dd1 = x - w1_ref[:, pl.ds(0, D)]
    dd2 = x - w2_ref[:, pl.ds(0, D)]
    e1 = jnp.sum(dd1 * dd1, axis=1)
    e2 = jnp.sum(dd2 * dd2, axis=1)
    i1 = i1_ref[:]
    i2 = i2_ref[:]
    use2 = (e2 < e1) | ((e2 == e1) & (i2 < i1))
    bmu = jnp.where(use2, i2, i1)
    rc_ref[0, :] = bmu // GRID_W
    rc_ref[1, :] = bmu % GRID_W
    qe_ref[:] = jnp.sqrt(jnp.where(use2, e2, e1))


def kernel(x, weights_map):
    w_flat = jnp.reshape(weights_map, (N_NEURONS, D))
    i1, i2 = pl.pallas_call(
        _top2_body,
        out_shape=(
            jax.ShapeDtypeStruct((B,), jnp.int32),
            jax.ShapeDtypeStruct((B,), jnp.int32),
        ),
    )(x, w_flat)
    w1p, w2p = _make_sc_gather()(w_flat, i1, i2)
    rc2, qe = pl.pallas_call(
        _finish_body,
        out_shape=(
            jax.ShapeDtypeStruct((2, B), jnp.int32),
            jax.ShapeDtypeStruct((B,), jnp.float32),
        ),
    )(x, w1p, w2p, i1, i2)
    return jnp.transpose(rc2), qe


# speedup vs baseline: 1.1195x; 1.0117x over previous
import functools

import jax
import jax.numpy as jnp
from jax import lax
from jax.experimental import pallas as pl
from jax.experimental.pallas import tpu as pltpu
from jax.experimental.pallas import tpu_sc as plsc

GRID_W = 64
N_NEURONS = 4096
B = 1024
D = 32

_NC = 2
_NS = 16
_NW = _NC * _NS
_RPW = B // _NW
_PAD = 128


def _bf16_split_dot(a, b):
    ah = a.astype(jnp.bfloat16)
    al = (a - ah.astype(jnp.float32)).astype(jnp.bfloat16)
    bh = b.astype(jnp.bfloat16)
    bl = (b - bh.astype(jnp.float32)).astype(jnp.bfloat16)
    a3 = jnp.concatenate([ah, ah, al], axis=1)
    b3 = jnp.concatenate([bh, bl, bh], axis=0)
    return jax.lax.dot_general(
        a3, b3, (((1,), (0,)), ((), ())), preferred_element_type=jnp.float32)


def _top2_body(x_ref, w_ref, i1_ref, i2_ref):
    x = x_ref[:, :]
    wt = jnp.transpose(w_ref[:, :])

    xw = _bf16_split_dot(x, wt)
    wn = jnp.sum(wt * wt, axis=0)
    score = wn[None, :] - 2.0 * xw

    col = jax.lax.broadcasted_iota(jnp.int32, score.shape, 1)
    i1 = jnp.argmin(score, axis=1).astype(jnp.int32)
    masked = jnp.where(col == i1[:, None], jnp.inf, score)
    i2 = jnp.argmin(masked, axis=1).astype(jnp.int32)

    i1_ref[:] = i1
    i2_ref[:] = i2


@functools.lru_cache(maxsize=1)
def _make_sc_gather():
    mesh = plsc.VectorSubcoreMesh(
        core_axis_name="c", subcore_axis_name="s",
        num_cores=_NC, num_subcores=_NS,
    )

    @functools.partial(
        pl.kernel,
        out_type=(
            jax.ShapeDtypeStruct((B, _PAD), jnp.float32),
            jax.ShapeDtypeStruct((B, _PAD), jnp.float32),
        ),
        mesh=mesh,
        compiler_params=pltpu.CompilerParams(use_tc_tiling_on_sc=False),
        scratch_types=[
            pltpu.VMEM((_RPW,), jnp.int32),
            pltpu.VMEM((_RPW,), jnp.int32),
            pltpu.VMEM((_RPW, D), jnp.float32),
            pltpu.VMEM((_RPW, D), jnp.float32),
            pltpu.VMEM((_RPW, _PAD), jnp.float32),
            pltpu.VMEM((_RPW, _PAD), jnp.float32),
            pltpu.SemaphoreType.DMA,
        ],
    )
    def _sc_gather(w_hbm, i1_hbm, i2_hbm, w1_hbm, w2_hbm,
                   i1_v, i2_v, w1_v, w2_v, w1p_v, w2p_v, sem):
        wid = lax.axis_index("s") * _NC + lax.axis_index("c")
        base = wid * _RPW
        pltpu.sync_copy(i1_hbm.at[pl.ds(base, _RPW)], i1_v)
        pltpu.sync_copy(i2_hbm.at[pl.ds(base, _RPW)], i2_v)
        cp1 = pltpu.async_copy(w_hbm.at[i1_v], w1_v, sem)
        cp2 = pltpu.async_copy(w_hbm.at[i2_v], w2_v, sem)
        cp1.wait()
        cp2.wait()
        pltpu.sync_copy(w1_v, w1_hbm.at[pl.ds(base, _RPW), pl.ds(0, D)])
        pltpu.sync_copy(w2_v, w2_hbm.at[pl.ds(base, _RPW), pl.ds(0, D)])

    return _sc_gather


def _finish_body(x_ref, w1_ref, w2_ref, i1_ref, i2_ref, rc_ref, qe_ref):
    x = x_ref[:, :]
    dd1 = x - w1_ref[:, pl.ds(0, D)]
    dd2 = x - w2_ref[:, pl.ds(0, D)]
    e1 = jnp.sum(dd1 * dd1, axis=1)
    e2 = jnp.sum(dd2 * dd2, axis=1)
    i1 = i1_ref[:]
    i2 = i2_ref[:]
    use2 = (e2 < e1) | ((e2 == e1) & (i2 < i1))
    bmu = jnp.where(use2, i2, i1)
    rc_ref[0, :] = bmu // GRID_W
    rc_ref[1, :] = bmu % GRID_W
    qe_ref[:] = jnp.sqrt(jnp.where(use2, e2, e1))


def kernel(x, weights_map):
    w_flat = jnp.reshape(weights_map, (N_NEURONS, D))
    i1, i2 = pl.pallas_call(
        _top2_body,
        out_shape=(
            jax.ShapeDtypeStruct((B,), jnp.int32),
            jax.ShapeDtypeStruct((B,), jnp.int32),
        ),
    )(x, w_flat)
    w1p, w2p = _make_sc_gather()(w_flat, i1, i2)
    rc2, qe = pl.pallas_call(
        _finish_body,
        out_shape=(
            jax.ShapeDtypeStruct((2, B), jnp.int32),
            jax.ShapeDtypeStruct((B,), jnp.float32),
        ),
    )(x, w1p, w2p, i1, i2)
    return jnp.transpose(rc2), qe
